# baseline (speedup 1.0000x reference)
import functools

import jax
import jax.numpy as jnp
from jax import lax
from jax.experimental import pallas as pl
from jax.experimental.pallas import tpu as pltpu
from jax.experimental.pallas import tpu_sc as plsc

N_EDGES = 6400000
N_SEG = 100000
NC, NS = 2, 16
NW = NC * NS
EDGES_PER_TILE = N_EDGES // NW
CW = 1920
NVEC = CW // 16
VUNROLL = 8
N_CHUNKS = EDGES_PER_TILE // CW
TAIL = EDGES_PER_TILE - N_CHUNKS * CW
ACC_PAD = 102400
SLICE = ACC_PAD // NS
MBLK = 400

_mesh = plsc.VectorSubcoreMesh(core_axis_name="c", subcore_axis_name="s")


@functools.partial(
    pl.kernel,
    out_type=jax.ShapeDtypeStruct((NC, ACC_PAD), jnp.float32),
    mesh=_mesh,
    scratch_types=[
        pltpu.VMEM((2, CW), jnp.int32),
        pltpu.VMEM((2, CW), jnp.float32),
        pltpu.VMEM((ACC_PAD,), jnp.float32),
        pltpu.VMEM((NS, MBLK), jnp.float32),
        pltpu.VMEM_SHARED((NS, NS, MBLK), jnp.float32),
        pltpu.SemaphoreType.DMA,
        pltpu.SemaphoreType.DMA,
        pltpu.SemaphoreType.DMA,
    ],
    compiler_params=pltpu.CompilerParams(use_tc_tiling_on_sc=False,
                                         needs_layout_passes=False),
)
def _seg_sum_sc(vals_hbm, ids_hbm, out_hbm, idx_b, val_b, acc, mbuf, spst,
                sem_i, sem_v, sem_m):
    c = lax.axis_index("c")
    s = lax.axis_index("s")
    w = c * NS + s

    iota = lax.iota(jnp.int32, 16)
    stride_iota = iota * NVEC
    z16 = jnp.zeros((16,), jnp.float32)

    def _zb(i, carry):
        acc[pl.ds(i * 16, 16)] = z16
        return carry

    lax.fori_loop(0, ACC_PAD // 16, _zb, 0, unroll=8)

    base = w * EDGES_PER_TILE

    def _stage(ci, b, n=CW):
        e0 = base + ci * CW
        pltpu.async_copy(ids_hbm.at[pl.ds(e0, n)], idx_b.at[b, pl.ds(0, n)],
                         sem_i)
        pltpu.async_copy(vals_hbm.at[pl.ds(e0, n)], val_b.at[b, pl.ds(0, n)],
                         sem_v)

    def _wait_stage(b, n=CW):
        pltpu.make_async_copy(
            ids_hbm.at[pl.ds(0, n)], idx_b.at[b, pl.ds(0, n)], sem_i).wait()
        pltpu.make_async_copy(
            vals_hbm.at[pl.ds(0, n)], val_b.at[b, pl.ds(0, n)], sem_v).wait()

    _stage(0, 0)
    _stage(1, 1)

    def _outer(ci0, carry):
        for b in range(2):
            ci = ci0 * 2 + b
            _wait_stage(b)
            ib = idx_b.at[b]
            vb = val_b.at[b]

            def _vec(vi, carry2):
                for u in range(VUNROLL):
                    idxv = stride_iota + (vi * VUNROLL + u)
                    d = plsc.load_gather(ib, [idxv])
                    x = plsc.load_gather(vb, [idxv])
                    plsc.addupdate_scatter(acc, [d], x)
                return carry2

            lax.fori_loop(0, NVEC // VUNROLL, _vec, 0)

            @pl.when(ci + 2 < N_CHUNKS)
            def _():
                _stage(ci + 2, b)
        return carry

    lax.fori_loop(0, N_CHUNKS // 2, _outer, 0)

    _stage(N_CHUNKS, 0, TAIL)
    _wait_stage(0, TAIL)
    for v in range(TAIL // 16):
        d = idx_b.at[0][pl.ds(v * 16, 16)]
        x = val_b.at[0][pl.ds(v * 16, 16)]
        plsc.addupdate_scatter(acc, [d], x)

    r0 = s * SLICE
    zf = jnp.zeros((16,), jnp.float32)

    def _round(t, carry):
        def _pub(o, cc):
            pltpu.async_copy(
                acc.at[pl.ds(o * SLICE + t * MBLK, MBLK)],
                spst.at[o, s], sem_m)
            return cc

        lax.fori_loop(0, NS, _pub, 0)

        def _pubw(o, cc):
            pltpu.make_async_copy(
                acc.at[pl.ds(0, MBLK)], spst.at[o, s], sem_m).wait()
            return cc

        lax.fori_loop(0, NS, _pubw, 0)
        plsc.subcore_barrier()
        pltpu.sync_copy(spst.at[s], mbuf)

        def _red(v, cc):
            tot = mbuf[0, pl.ds(v * 16, 16)]
            for j in range(1, NS):
                tot = tot + mbuf[j, pl.ds(v * 16, 16)]
            acc[pl.ds(t * MBLK + v * 16, 16)] = tot
            return cc

        lax.fori_loop(0, MBLK // 16, _red, 0, unroll=5)
        plsc.subcore_barrier()
        return carry

    lax.fori_loop(0, SLICE // MBLK, _round, 0)

    pltpu.sync_copy(acc.at[pl.ds(0, SLICE)],
                    out_hbm.at[c, pl.ds(r0, SLICE)])


def _combine_body(p_ref, o_ref):
    o_ref[...] = jnp.sum(p_ref[...], axis=0, keepdims=True)


def kernel(arg0_1, arg1_1):
    vals = arg0_1.reshape(N_EDGES)
    ids = arg1_1.astype(jnp.int32).reshape(N_EDGES)
    partials = _seg_sum_sc(vals, ids)
    summed = pl.pallas_call(
        _combine_body,
        out_shape=jax.ShapeDtypeStruct((1, ACC_PAD), jnp.float32),
    )(partials)
    return (summed[0, :N_SEG].reshape(N_SEG, 1),)


# speedup vs baseline: 1.1883x; 1.1883x over previous
import functools

import jax
import jax.numpy as jnp
from jax import lax
from jax.experimental import pallas as pl
from jax.experimental.pallas import tpu as pltpu
from jax.experimental.pallas import tpu_sc as plsc

N_EDGES = 6400000
N_SEG = 100000
NC, NS = 2, 16
NW = NC * NS
EDGES_PER_TILE = N_EDGES // NW
CW = 2000
NVEC = CW // 16
VUNROLL = 5
N_CHUNKS = EDGES_PER_TILE // CW
TAIL = EDGES_PER_TILE - N_CHUNKS * CW
ACC_PAD = 102400
SLICE = ACC_PAD // NS
MBLK = 400

_mesh = plsc.VectorSubcoreMesh(core_axis_name="c", subcore_axis_name="s")


@functools.partial(
    pl.kernel,
    out_type=jax.ShapeDtypeStruct((NC, ACC_PAD), jnp.float32),
    mesh=_mesh,
    scratch_types=[
        pltpu.VMEM((2, CW), jnp.int32),
        pltpu.VMEM((2, CW), jnp.float32),
        pltpu.VMEM((ACC_PAD,), jnp.float32),
        pltpu.VMEM((NS, MBLK), jnp.float32),
        pltpu.VMEM_SHARED((NS, NS, MBLK), jnp.float32),
        pltpu.SemaphoreType.DMA,
        pltpu.SemaphoreType.DMA,
        pltpu.SemaphoreType.DMA,
    ],
    compiler_params=pltpu.CompilerParams(use_tc_tiling_on_sc=False,
                                         needs_layout_passes=False),
)
def _seg_sum_sc(vals_hbm, ids_hbm, out_hbm, idx_b, val_b, acc, mbuf, spst,
                sem_i, sem_v, sem_m):
    c = lax.axis_index("c")
    s = lax.axis_index("s")
    w = c * NS + s

    iota = lax.iota(jnp.int32, 16)
    stride_iota = iota * NVEC
    z16 = jnp.zeros((16,), jnp.float32)

    def _zb(i, carry):
        acc[pl.ds(i * 16, 16)] = z16
        return carry

    lax.fori_loop(0, ACC_PAD // 16, _zb, 0, unroll=8)

    base = w * EDGES_PER_TILE

    def _stage(ci, b, n=CW):
        e0 = base + ci * CW
        pltpu.async_copy(ids_hbm.at[pl.ds(e0, n)], idx_b.at[b, pl.ds(0, n)],
                         sem_i)
        pltpu.async_copy(vals_hbm.at[pl.ds(e0, n)], val_b.at[b, pl.ds(0, n)],
                         sem_v)

    def _wait_stage(b, n=CW):
        pltpu.make_async_copy(
            ids_hbm.at[pl.ds(0, n)], idx_b.at[b, pl.ds(0, n)], sem_i).wait()
        pltpu.make_async_copy(
            vals_hbm.at[pl.ds(0, n)], val_b.at[b, pl.ds(0, n)], sem_v).wait()

    _stage(0, 0)
    _stage(1, 1)

    def _outer(ci0, carry):
        for b in range(2):
            ci = ci0 * 2 + b
            _wait_stage(b)
            ib = idx_b.at[b]
            vb = val_b.at[b]

            def _vec(vi, carry2):
                for u in range(VUNROLL):
                    idxv = stride_iota + (vi * VUNROLL + u)
                    d = plsc.load_gather(ib, [idxv])
                    x = plsc.load_gather(vb, [idxv])
                    plsc.addupdate_scatter(acc, [d], x)
                return carry2

            lax.fori_loop(0, NVEC // VUNROLL, _vec, 0)

            @pl.when(ci + 2 < N_CHUNKS)
            def _():
                _stage(ci + 2, b)
        return carry

    lax.fori_loop(0, N_CHUNKS // 2, _outer, 0)

    r0 = s * SLICE
    zf = jnp.zeros((16,), jnp.float32)

    def _round(t, carry):
        def _pub(o, cc):
            pltpu.async_copy(
                acc.at[pl.ds(o * SLICE + t * MBLK, MBLK)],
                spst.at[o, s], sem_m)
            return cc

        lax.fori_loop(0, NS, _pub, 0)

        def _pubw(o, cc):
            pltpu.make_async_copy(
                acc.at[pl.ds(0, MBLK)], spst.at[o, s], sem_m).wait()
            return cc

        lax.fori_loop(0, NS, _pubw, 0)
        plsc.subcore_barrier()
        pltpu.sync_copy(spst.at[s], mbuf)

        def _red(v, cc):
            tot = mbuf[0, pl.ds(v * 16, 16)]
            for j in range(1, NS):
                tot = tot + mbuf[j, pl.ds(v * 16, 16)]
            acc[pl.ds(t * MBLK + v * 16, 16)] = tot
            return cc

        lax.fori_loop(0, MBLK // 16, _red, 0, unroll=5)
        plsc.subcore_barrier()
        return carry

    lax.fori_loop(0, SLICE // MBLK, _round, 0)

    pltpu.sync_copy(acc.at[pl.ds(0, SLICE)],
                    out_hbm.at[c, pl.ds(r0, SLICE)])


def _combine_body(p_ref, o_ref):
    o_ref[...] = jnp.sum(p_ref[...], axis=0, keepdims=True)


def kernel(arg0_1, arg1_1):
    vals = arg0_1.reshape(N_EDGES)
    ids = arg1_1.astype(jnp.int32).reshape(N_EDGES)
    partials = _seg_sum_sc(vals, ids)
    summed = pl.pallas_call(
        _combine_body,
        out_shape=jax.ShapeDtypeStruct((1, ACC_PAD), jnp.float32),
    )(partials)
    return (summed[0, :N_SEG].reshape(N_SEG, 1),)


# speedup vs baseline: 1.1917x; 1.0029x over previous
import functools

import jax
import jax.numpy as jnp
from jax import lax
from jax.experimental import pallas as pl
from jax.experimental.pallas import tpu as pltpu
from jax.experimental.pallas import tpu_sc as plsc

N_EDGES = 6400000
N_SEG = 100000
NC, NS = 2, 16
NW = NC * NS
EDGES_PER_TILE = N_EDGES // NW
CW = 2000
SS = 608
NVEC = (CW - SS) // 16
VUNROLL = 3
N_CHUNKS = EDGES_PER_TILE // CW
TAIL = EDGES_PER_TILE - N_CHUNKS * CW
ACC_PAD = 102400
SLICE = ACC_PAD // NS
MBLK = 400

_mesh = plsc.VectorSubcoreMesh(core_axis_name="c", subcore_axis_name="s")


@functools.partial(
    pl.kernel,
    out_type=jax.ShapeDtypeStruct((NC, ACC_PAD), jnp.float32),
    mesh=_mesh,
    scratch_types=[
        pltpu.VMEM((2, CW), jnp.int32),
        pltpu.VMEM((2, CW), jnp.float32),
        pltpu.VMEM((ACC_PAD,), jnp.float32),
        pltpu.VMEM((NS + 1, MBLK), jnp.float32),
        pltpu.VMEM_SHARED((NS, NS, MBLK), jnp.float32),
        pltpu.VMEM_SHARED((ACC_PAD,), jnp.float32),
        pltpu.SemaphoreType.DMA,
        pltpu.SemaphoreType.DMA,
        pltpu.SemaphoreType.DMA,
        pltpu.SemaphoreType.DMA,
    ],
    compiler_params=pltpu.CompilerParams(use_tc_tiling_on_sc=False,
                                         needs_layout_passes=False),
)
def _seg_sum_sc(vals_hbm, ids_hbm, out_hbm, idx_b, val_b, acc, mbuf, spst,
                spacc, sem_i, sem_v, sem_m, sem_s):
    c = lax.axis_index("c")
    s = lax.axis_index("s")
    w = c * NS + s

    iota = lax.iota(jnp.int32, 16)
    stride_iota = iota * NVEC
    z16 = jnp.zeros((16,), jnp.float32)

    def _zb(i, carry):
        acc[pl.ds(i * 16, 16)] = z16
        return carry

    lax.fori_loop(0, ACC_PAD // 16, _zb, 0, unroll=8)
    pltpu.sync_copy(acc.at[pl.ds(0, SLICE)],
                    spacc.at[pl.ds(s * SLICE, SLICE)])
    plsc.subcore_barrier()

    base = w * EDGES_PER_TILE

    def _stage(ci, b, n=CW):
        e0 = base + ci * CW
        pltpu.async_copy(ids_hbm.at[pl.ds(e0, n)], idx_b.at[b, pl.ds(0, n)],
                         sem_i)
        pltpu.async_copy(vals_hbm.at[pl.ds(e0, n)], val_b.at[b, pl.ds(0, n)],
                         sem_v)

    def _wait_stage(b, n=CW):
        pltpu.make_async_copy(
            ids_hbm.at[pl.ds(0, n)], idx_b.at[b, pl.ds(0, n)], sem_i).wait()
        pltpu.make_async_copy(
            vals_hbm.at[pl.ds(0, n)], val_b.at[b, pl.ds(0, n)], sem_v).wait()

    _stage(0, 0)
    _stage(1, 1)

    def _outer(ci0, carry):
        for b in range(2):
            ci = ci0 * 2 + b
            _wait_stage(b)
            ib = idx_b.at[b]
            vb = val_b.at[b]

            pltpu.async_copy(vb.at[pl.ds(0, SS)],
                             spacc.at[ib.at[pl.ds(0, SS)]], sem_s, add=True)

            def _vec(vi, carry2):
                for u in range(VUNROLL):
                    idxv = stride_iota + (SS + vi * VUNROLL + u)
                    d = plsc.load_gather(ib, [idxv])
                    x = plsc.load_gather(vb, [idxv])
                    plsc.addupdate_scatter(acc, [d], x)
                return carry2

            lax.fori_loop(0, NVEC // VUNROLL, _vec, 0)

            pltpu.make_async_copy(
                vb.at[pl.ds(0, SS)],
                spacc.at[ib.at[pl.ds(0, SS)]], sem_s).wait()

            @pl.when(ci + 2 < N_CHUNKS)
            def _():
                _stage(ci + 2, b)
        return carry

    lax.fori_loop(0, N_CHUNKS // 2, _outer, 0)

    r0 = s * SLICE
    zf = jnp.zeros((16,), jnp.float32)

    def _round(t, carry):
        def _pub(o, cc):
            pltpu.async_copy(
                acc.at[pl.ds(o * SLICE + t * MBLK, MBLK)],
                spst.at[o, s], sem_m)
            return cc

        lax.fori_loop(0, NS, _pub, 0)

        def _pubw(o, cc):
            pltpu.make_async_copy(
                acc.at[pl.ds(0, MBLK)], spst.at[o, s], sem_m).wait()
            return cc

        lax.fori_loop(0, NS, _pubw, 0)
        plsc.subcore_barrier()
        pltpu.sync_copy(spst.at[s], mbuf.at[pl.ds(0, NS)])
        pltpu.sync_copy(spacc.at[pl.ds(r0 + t * MBLK, MBLK)], mbuf.at[NS])

        def _red(v, cc):
            tot = mbuf[0, pl.ds(v * 16, 16)]
            for j in range(1, NS + 1):
                tot = tot + mbuf[j, pl.ds(v * 16, 16)]
            acc[pl.ds(t * MBLK + v * 16, 16)] = tot
            return cc

        lax.fori_loop(0, MBLK // 16, _red, 0, unroll=5)
        plsc.subcore_barrier()
        return carry

    lax.fori_loop(0, SLICE // MBLK, _round, 0)

    pltpu.sync_copy(acc.at[pl.ds(0, SLICE)],
                    out_hbm.at[c, pl.ds(r0, SLICE)])


def _combine_body(p_ref, o_ref):
    o_ref[...] = jnp.sum(p_ref[...], axis=0, keepdims=True)


def kernel(arg0_1, arg1_1):
    vals = arg0_1.reshape(N_EDGES)
    ids = arg1_1.astype(jnp.int32).reshape(N_EDGES)
    partials = _seg_sum_sc(vals, ids)
    summed = pl.pallas_call(
        _combine_body,
        out_shape=jax.ShapeDtypeStruct((1, ACC_PAD), jnp.float32),
    )(partials)
    return (summed[0, :N_SEG].reshape(N_SEG, 1),)
